# baseline (speedup 1.0000x reference)
import jax
import jax.numpy as jnp
from jax.experimental import pallas as pl
from jax.experimental.pallas import tpu as pltpu

_N_VIEW = 4
_N_POINT = 1024
_T = 0.01
_PAIRS_I = (0, 0, 0, 1, 1, 2)
_PAIRS_J = (1, 2, 3, 2, 3, 3)


def _pair_kernel(pi_ref, pjt_ref,
                 wf1_ref, bf1r_ref, wf2_ref, bf2r_ref,
                 wc1_ref, bc1r_ref, wc2_ref, bc2r_ref,
                 wf1t_ref, bf1c_ref, wf2t_ref, bf2c_ref,
                 wc1t_ref, bc1c_ref, wc2t_ref,
                 out_ref):
    f32 = jnp.float32
    pc_i = pi_ref[0]
    pc_jt = pjt_ref[0]

    h_i = jnp.tanh(jnp.dot(pc_i, wf1_ref[...], preferred_element_type=f32)
                   + bf1r_ref[...])
    a_i = pc_i + jnp.dot(h_i, wf2_ref[...], preferred_element_type=f32) \
        + bf2r_ref[...]
    hc_i = jnp.tanh(jnp.dot(a_i, wc1_ref[...], preferred_element_type=f32)
                    + bc1r_ref[...])
    w_i = jax.nn.sigmoid(
        jnp.dot(hc_i, wc2_ref[...], preferred_element_type=f32)
        + bc2r_ref[...])

    h_jt = jnp.tanh(jnp.dot(wf1t_ref[...], pc_jt, preferred_element_type=f32)
                    + bf1c_ref[...])
    b_jt = pc_jt + jnp.dot(wf2t_ref[...], h_jt, preferred_element_type=f32) \
        + bf2c_ref[...]
    hc_jt = jnp.tanh(jnp.dot(wc1t_ref[...], b_jt, preferred_element_type=f32)
                     + bc1c_ref[...])
    w_j = jax.nn.sigmoid(
        jnp.dot(wc2t_ref[...], hc_jt, preferred_element_type=f32)
        + bc2r_ref[...])

    n2_ai = jnp.sum(a_i * a_i, axis=1, keepdims=True) + 0.5e-12
    n2_pi = jnp.sum(pc_i * pc_i, axis=1, keepdims=True) + 0.5e-12
    n2_pj = jnp.sum(pc_jt * pc_jt, axis=0, keepdims=True) + 0.5e-12
    n2_bj = jnp.sum(b_jt * b_jt, axis=0, keepdims=True) + 0.5e-12
    hp = jax.lax.Precision.HIGHEST
    cross12 = jnp.dot(a_i + a_i, pc_jt, preferred_element_type=f32,
                      precision=hp)
    cross21 = jnp.dot(pc_i + pc_i, b_jt, preferred_element_type=f32,
                      precision=hp)
    d12 = jnp.sqrt(jnp.abs((n2_ai + n2_pj) - cross12))
    d21 = jnp.sqrt(jnp.abs((n2_pi + n2_bj) - cross21))

    u = w_j / (w_i + w_j)
    blend = d12 + (d21 - d12) * u
    mb = jnp.min(blend, axis=1, keepdims=True) * (1.0 / _T)
    e = jnp.exp(mb - blend * (1.0 / _T))
    out_ref[0] = e * (1.0 / jnp.sum(e, axis=1, keepdims=True))


def kernel(xyz, Wf1, bf1, Wf2, bf2, Wc1, bc1, Wc2, bc2):
    x = xyz[0]
    pi = jnp.stack([x[i] for i in _PAIRS_I])
    pjt = jnp.stack([x[j].T for j in _PAIRS_J])

    full = lambda shape: pl.BlockSpec(shape, lambda p: (0,) * len(shape))
    in_specs = [
        pl.BlockSpec((1, _N_POINT, 3), lambda p: (p, 0, 0)),
        pl.BlockSpec((1, 3, _N_POINT), lambda p: (p, 0, 0)),
        full((3, 64)), full((1, 64)), full((64, 3)), full((1, 3)),
        full((3, 64)), full((1, 64)), full((64, 1)), full((1, 1)),
        full((64, 3)), full((64, 1)), full((3, 64)), full((3, 1)),
        full((64, 3)), full((64, 1)), full((1, 64)),
    ]
    out = pl.pallas_call(
        _pair_kernel,
        grid=(6,),
        in_specs=in_specs,
        out_specs=pl.BlockSpec((1, _N_POINT, _N_POINT), lambda p: (p, 0, 0)),
        out_shape=jax.ShapeDtypeStruct((6, _N_POINT, _N_POINT), jnp.float32),
        compiler_params=pltpu.CompilerParams(
            dimension_semantics=("arbitrary",)),
    )(
        pi, pjt,
        Wf1, bf1.reshape(1, 64), Wf2, bf2.reshape(1, 3),
        Wc1, bc1.reshape(1, 64), Wc2, bc2.reshape(1, 1),
        Wf1.T, bf1.reshape(64, 1), Wf2.T, bf2.reshape(3, 1),
        Wc1.T, bc1.reshape(64, 1), Wc2.T,
    )
    return out.reshape(6, 1, _N_POINT, _N_POINT)


# speedup vs baseline: 1.2292x; 1.2292x over previous
import jax
import jax.numpy as jnp
from jax.experimental import pallas as pl
from jax.experimental.pallas import tpu as pltpu

_N_VIEW = 4
_N_POINT = 1024
_T = 0.01
_PAIRS_I = (0, 0, 0, 1, 1, 2)
_PAIRS_J = (1, 2, 3, 2, 3, 3)


def _pair_kernel(pi_ref, pjt_ref,
                 wf1_ref, bf1r_ref, wf2_ref, bf2r_ref,
                 wc1_ref, bc1r_ref, wc2_ref, bc2r_ref,
                 wf1t_ref, bf1c_ref, wf2t_ref, bf2c_ref,
                 wc1t_ref, bc1c_ref, wc2t_ref,
                 out_ref):
    f32 = jnp.float32
    pc_i = pi_ref[0]
    pc_jt = pjt_ref[0]

    h_i = jnp.tanh(jnp.dot(pc_i, wf1_ref[...], preferred_element_type=f32)
                   + bf1r_ref[...])
    a_i = pc_i + jnp.dot(h_i, wf2_ref[...], preferred_element_type=f32) \
        + bf2r_ref[...]
    hc_i = jnp.tanh(jnp.dot(a_i, wc1_ref[...], preferred_element_type=f32)
                    + bc1r_ref[...])
    w_i = jax.nn.sigmoid(
        jnp.dot(hc_i, wc2_ref[...], preferred_element_type=f32)
        + bc2r_ref[...])

    h_jt = jnp.tanh(jnp.dot(wf1t_ref[...], pc_jt, preferred_element_type=f32)
                    + bf1c_ref[...])
    b_jt = pc_jt + jnp.dot(wf2t_ref[...], h_jt, preferred_element_type=f32) \
        + bf2c_ref[...]
    hc_jt = jnp.tanh(jnp.dot(wc1t_ref[...], b_jt, preferred_element_type=f32)
                     + bc1c_ref[...])
    w_j = jax.nn.sigmoid(
        jnp.dot(wc2t_ref[...], hc_jt, preferred_element_type=f32)
        + bc2r_ref[...])

    dx12 = a_i[:, 0:1] - pc_jt[0:1, :]
    dy12 = a_i[:, 1:2] - pc_jt[1:2, :]
    dz12 = a_i[:, 2:3] - pc_jt[2:3, :]
    d12 = jnp.sqrt(dx12 * dx12 + dy12 * dy12 + dz12 * dz12)
    dx21 = pc_i[:, 0:1] - b_jt[0:1, :]
    dy21 = pc_i[:, 1:2] - b_jt[1:2, :]
    dz21 = pc_i[:, 2:3] - b_jt[2:3, :]
    d21 = jnp.sqrt(dx21 * dx21 + dy21 * dy21 + dz21 * dz21)

    u = w_j / (w_i + w_j)
    blend = d12 + (d21 - d12) * u
    mb = jnp.min(blend, axis=1, keepdims=True) * (1.0 / _T)
    e = jnp.exp(mb - blend * (1.0 / _T))
    out_ref[0, 0] = e * (1.0 / jnp.sum(e, axis=1, keepdims=True))


def kernel(xyz, Wf1, bf1, Wf2, bf2, Wc1, bc1, Wc2, bc2):
    x = xyz[0]
    pi = jnp.stack([x[i] for i in _PAIRS_I])
    pjt = jnp.stack([x[j].T for j in _PAIRS_J])

    full = lambda shape: pl.BlockSpec(shape, lambda p: (0,) * len(shape))
    in_specs = [
        pl.BlockSpec((1, _N_POINT, 3), lambda p: (p, 0, 0)),
        pl.BlockSpec((1, 3, _N_POINT), lambda p: (p, 0, 0)),
        full((3, 64)), full((1, 64)), full((64, 3)), full((1, 3)),
        full((3, 64)), full((1, 64)), full((64, 1)), full((1, 1)),
        full((64, 3)), full((64, 1)), full((3, 64)), full((3, 1)),
        full((64, 3)), full((64, 1)), full((1, 64)),
    ]
    out = pl.pallas_call(
        _pair_kernel,
        grid=(6,),
        in_specs=in_specs,
        out_specs=pl.BlockSpec((1, 1, _N_POINT, _N_POINT),
                               lambda p: (p, 0, 0, 0)),
        out_shape=jax.ShapeDtypeStruct((6, 1, _N_POINT, _N_POINT),
                                       jnp.float32),
        compiler_params=pltpu.CompilerParams(
            dimension_semantics=("arbitrary",)),
    )(
        pi, pjt,
        Wf1, bf1.reshape(1, 64), Wf2, bf2.reshape(1, 3),
        Wc1, bc1.reshape(1, 64), Wc2, bc2.reshape(1, 1),
        Wf1.T, bf1.reshape(64, 1), Wf2.T, bf2.reshape(3, 1),
        Wc1.T, bc1.reshape(64, 1), Wc2.T,
    )
    return out


# speedup vs baseline: 1.4138x; 1.1502x over previous
import jax
import jax.numpy as jnp
from jax.experimental import pallas as pl
from jax.experimental.pallas import tpu as pltpu

_N_VIEW = 4
_N_POINT = 1024
_T = 0.01
_PAIRS_I = (0, 0, 0, 1, 1, 2)
_PAIRS_J = (1, 2, 3, 2, 3, 3)


def _pair_kernel(pi_ref, pjt_ref,
                 wf1_ref, bf1r_ref, wf2_ref, bf2r_ref,
                 wc1_ref, bc1r_ref, wc2_ref, bc2r_ref,
                 wf1t_ref, bf1c_ref, wf2t_ref, bf2c_ref,
                 wc1t_ref, bc1c_ref, wc2t_ref,
                 out_ref):
    f32 = jnp.float32
    pc_i = pi_ref[0]
    pc_jt = pjt_ref[0]

    h_i = jnp.tanh(jnp.dot(pc_i, wf1_ref[...], preferred_element_type=f32)
                   + bf1r_ref[...])
    a_i = pc_i + jnp.dot(h_i, wf2_ref[...], preferred_element_type=f32) \
        + bf2r_ref[...]
    hc_i = jnp.tanh(jnp.dot(a_i, wc1_ref[...], preferred_element_type=f32)
                    + bc1r_ref[...])
    w_i = jax.nn.sigmoid(
        jnp.dot(hc_i, wc2_ref[...], preferred_element_type=f32)
        + bc2r_ref[...])

    h_jt = jnp.tanh(jnp.dot(wf1t_ref[...], pc_jt, preferred_element_type=f32)
                    + bf1c_ref[...])
    b_jt = pc_jt + jnp.dot(wf2t_ref[...], h_jt, preferred_element_type=f32) \
        + bf2c_ref[...]
    hc_jt = jnp.tanh(jnp.dot(wc1t_ref[...], b_jt, preferred_element_type=f32)
                     + bc1c_ref[...])
    w_j = jax.nn.sigmoid(
        jnp.dot(wc2t_ref[...], hc_jt, preferred_element_type=f32)
        + bc2r_ref[...])

    c = 1.4426950408889634 / _T
    aic = a_i * c
    pic = pc_i * c
    pjtc = pc_jt * c
    bjtc = b_jt * c
    dx12 = aic[:, 0:1] - pjtc[0:1, :]
    dy12 = aic[:, 1:2] - pjtc[1:2, :]
    dz12 = aic[:, 2:3] - pjtc[2:3, :]
    s12 = dx12 * dx12 + dy12 * dy12 + dz12 * dz12 + 1e-24
    d12 = s12 * jax.lax.rsqrt(s12)
    dx21 = pic[:, 0:1] - bjtc[0:1, :]
    dy21 = pic[:, 1:2] - bjtc[1:2, :]
    dz21 = pic[:, 2:3] - bjtc[2:3, :]
    s21 = dx21 * dx21 + dy21 * dy21 + dz21 * dz21 + 1e-24
    d21 = s21 * jax.lax.rsqrt(s21)

    u = w_j / (w_i + w_j)
    blend = d12 + (d21 - d12) * u
    mb = jnp.min(blend, axis=1, keepdims=True)
    e = jnp.exp2(mb - blend)
    out_ref[0, 0] = e * (1.0 / jnp.sum(e, axis=1, keepdims=True))


def kernel(xyz, Wf1, bf1, Wf2, bf2, Wc1, bc1, Wc2, bc2):
    x = xyz[0]
    pi = jnp.stack([x[i] for i in _PAIRS_I])
    pjt = jnp.stack([x[j].T for j in _PAIRS_J])

    full = lambda shape: pl.BlockSpec(shape, lambda p: (0,) * len(shape))
    in_specs = [
        pl.BlockSpec((1, _N_POINT, 3), lambda p: (p, 0, 0)),
        pl.BlockSpec((1, 3, _N_POINT), lambda p: (p, 0, 0)),
        full((3, 64)), full((1, 64)), full((64, 3)), full((1, 3)),
        full((3, 64)), full((1, 64)), full((64, 1)), full((1, 1)),
        full((64, 3)), full((64, 1)), full((3, 64)), full((3, 1)),
        full((64, 3)), full((64, 1)), full((1, 64)),
    ]
    out = pl.pallas_call(
        _pair_kernel,
        grid=(6,),
        in_specs=in_specs,
        out_specs=pl.BlockSpec((1, 1, _N_POINT, _N_POINT),
                               lambda p: (p, 0, 0, 0)),
        out_shape=jax.ShapeDtypeStruct((6, 1, _N_POINT, _N_POINT),
                                       jnp.float32),
        compiler_params=pltpu.CompilerParams(
            dimension_semantics=("arbitrary",)),
    )(
        pi, pjt,
        Wf1, bf1.reshape(1, 64), Wf2, bf2.reshape(1, 3),
        Wc1, bc1.reshape(1, 64), Wc2, bc2.reshape(1, 1),
        Wf1.T, bf1.reshape(64, 1), Wf2.T, bf2.reshape(3, 1),
        Wc1.T, bc1.reshape(64, 1), Wc2.T,
    )
    return out


# speedup vs baseline: 1.4297x; 1.0112x over previous
import jax
import jax.numpy as jnp
from jax.experimental import pallas as pl
from jax.experimental.pallas import tpu as pltpu

_N_VIEW = 4
_N_POINT = 1024
_T = 0.01
_PAIRS_I = (0, 0, 0, 1, 1, 2)
_PAIRS_J = (1, 2, 3, 2, 3, 3)


def _pair_kernel(pi_ref, pjt_ref,
                 wf1_ref, bf1r_ref, wf2_ref, bf2r_ref,
                 wc1_ref, bc1r_ref, wc2_ref, bc2r_ref,
                 wf1t_ref, bf1c_ref, wf2t_ref, bf2c_ref,
                 wc1t_ref, bc1c_ref, wc2t_ref,
                 out_ref):
    for q in range(2):
        _one_pair(pi_ref[q], pjt_ref[q],
                  wf1_ref, bf1r_ref, wf2_ref, bf2r_ref,
                  wc1_ref, bc1r_ref, wc2_ref, bc2r_ref,
                  wf1t_ref, bf1c_ref, wf2t_ref, bf2c_ref,
                  wc1t_ref, bc1c_ref, wc2t_ref,
                  out_ref, q)


def _one_pair(pc_i, pc_jt,
              wf1_ref, bf1r_ref, wf2_ref, bf2r_ref,
              wc1_ref, bc1r_ref, wc2_ref, bc2r_ref,
              wf1t_ref, bf1c_ref, wf2t_ref, bf2c_ref,
              wc1t_ref, bc1c_ref, wc2t_ref,
              out_ref, q):
    f32 = jnp.float32

    h_i = jnp.tanh(jnp.dot(pc_i, wf1_ref[...], preferred_element_type=f32)
                   + bf1r_ref[...])
    a_i = pc_i + jnp.dot(h_i, wf2_ref[...], preferred_element_type=f32) \
        + bf2r_ref[...]
    hc_i = jnp.tanh(jnp.dot(a_i, wc1_ref[...], preferred_element_type=f32)
                    + bc1r_ref[...])
    w_i = jax.nn.sigmoid(
        jnp.dot(hc_i, wc2_ref[...], preferred_element_type=f32)
        + bc2r_ref[...])

    h_jt = jnp.tanh(jnp.dot(wf1t_ref[...], pc_jt, preferred_element_type=f32)
                    + bf1c_ref[...])
    b_jt = pc_jt + jnp.dot(wf2t_ref[...], h_jt, preferred_element_type=f32) \
        + bf2c_ref[...]
    hc_jt = jnp.tanh(jnp.dot(wc1t_ref[...], b_jt, preferred_element_type=f32)
                     + bc1c_ref[...])
    w_j = jax.nn.sigmoid(
        jnp.dot(wc2t_ref[...], hc_jt, preferred_element_type=f32)
        + bc2r_ref[...])

    c = 1.4426950408889634 / _T
    aic = a_i * c
    pic = pc_i * c
    pjtc = pc_jt * c
    bjtc = b_jt * c
    dx12 = aic[:, 0:1] - pjtc[0:1, :]
    dy12 = aic[:, 1:2] - pjtc[1:2, :]
    dz12 = aic[:, 2:3] - pjtc[2:3, :]
    s12 = dx12 * dx12 + dy12 * dy12 + dz12 * dz12 + 1e-24
    d12 = s12 * jax.lax.rsqrt(s12)
    dx21 = pic[:, 0:1] - bjtc[0:1, :]
    dy21 = pic[:, 1:2] - bjtc[1:2, :]
    dz21 = pic[:, 2:3] - bjtc[2:3, :]
    s21 = dx21 * dx21 + dy21 * dy21 + dz21 * dz21 + 1e-24
    d21 = s21 * jax.lax.rsqrt(s21)

    u = w_j / (w_i + w_j)
    blend = d12 + (d21 - d12) * u
    mb = jnp.min(blend, axis=1, keepdims=True)
    e = jnp.exp2(mb - blend)
    out_ref[q, 0] = e * (1.0 / jnp.sum(e, axis=1, keepdims=True))


def kernel(xyz, Wf1, bf1, Wf2, bf2, Wc1, bc1, Wc2, bc2):
    x = xyz[0]
    pi = jnp.stack([x[i] for i in _PAIRS_I])
    pjt = jnp.stack([x[j].T for j in _PAIRS_J])

    full = lambda shape: pl.BlockSpec(shape, lambda p: (0,) * len(shape))
    in_specs = [
        pl.BlockSpec((2, _N_POINT, 3), lambda p: (p, 0, 0)),
        pl.BlockSpec((2, 3, _N_POINT), lambda p: (p, 0, 0)),
        full((3, 64)), full((1, 64)), full((64, 3)), full((1, 3)),
        full((3, 64)), full((1, 64)), full((64, 1)), full((1, 1)),
        full((64, 3)), full((64, 1)), full((3, 64)), full((3, 1)),
        full((64, 3)), full((64, 1)), full((1, 64)),
    ]
    out = pl.pallas_call(
        _pair_kernel,
        grid=(3,),
        in_specs=in_specs,
        out_specs=pl.BlockSpec((2, 1, _N_POINT, _N_POINT),
                               lambda p: (p, 0, 0, 0)),
        out_shape=jax.ShapeDtypeStruct((6, 1, _N_POINT, _N_POINT),
                                       jnp.float32),
        compiler_params=pltpu.CompilerParams(
            dimension_semantics=("arbitrary",)),
    )(
        pi, pjt,
        Wf1, bf1.reshape(1, 64), Wf2, bf2.reshape(1, 3),
        Wc1, bc1.reshape(1, 64), Wc2, bc2.reshape(1, 1),
        Wf1.T, bf1.reshape(64, 1), Wf2.T, bf2.reshape(3, 1),
        Wc1.T, bc1.reshape(64, 1), Wc2.T,
    )
    return out


# speedup vs baseline: 1.4311x; 1.0010x over previous
import jax
import jax.numpy as jnp
from jax.experimental import pallas as pl
from jax.experimental.pallas import tpu as pltpu

_N_POINT = 1024
_T = 0.01
_VIDX = ((0, 1, 0, 2), (0, 3, 1, 2), (1, 3, 2, 3))


def _pair_kernel(vidx_ref, xi0_ref, xj0_ref, xi1_ref, xj1_ref,
                 wf1_ref, bf1r_ref, bf1c_ref, wf2_ref, bf2r_ref, bf2c_ref,
                 wc1_ref, bc1r_ref, bc1c_ref, wc2_ref, bc2r_ref,
                 out_ref):
    del vidx_ref
    wf1t = wf1_ref[...].T
    wf2t = wf2_ref[...].T
    wc1t = wc1_ref[...].T
    wc2t = wc2_ref[...].T
    for q, (xi_ref, xj_ref) in enumerate(((xi0_ref, xj0_ref),
                                          (xi1_ref, xj1_ref))):
        _one_pair(xi_ref[0, 0], xj_ref[0, 0],
                  wf1_ref[...], bf1r_ref[...], wf2_ref[...], bf2r_ref[...],
                  wc1_ref[...], bc1r_ref[...], wc2_ref[...], bc2r_ref[...],
                  wf1t, bf1c_ref[...], wf2t, bf2c_ref[...],
                  wc1t, bc1c_ref[...], wc2t,
                  out_ref, q)


def _one_pair(pc_i, pc_j,
              wf1, bf1r, wf2, bf2r, wc1, bc1r, wc2, bc2r,
              wf1t, bf1c, wf2t, bf2c, wc1t, bc1c, wc2t,
              out_ref, q):
    f32 = jnp.float32
    pc_jt = pc_j.T

    h_i = jnp.tanh(jnp.dot(pc_i, wf1, preferred_element_type=f32)
                   + bf1r)
    a_i = pc_i + jnp.dot(h_i, wf2, preferred_element_type=f32) \
        + bf2r
    hc_i = jnp.tanh(jnp.dot(a_i, wc1, preferred_element_type=f32)
                    + bc1r)
    w_i = jax.nn.sigmoid(
        jnp.dot(hc_i, wc2, preferred_element_type=f32)
        + bc2r)

    h_jt = jnp.tanh(jnp.dot(wf1t, pc_jt, preferred_element_type=f32)
                    + bf1c)
    b_jt = pc_jt + jnp.dot(wf2t, h_jt, preferred_element_type=f32) \
        + bf2c
    hc_jt = jnp.tanh(jnp.dot(wc1t, b_jt, preferred_element_type=f32)
                     + bc1c)
    w_j = jax.nn.sigmoid(
        jnp.dot(wc2t, hc_jt, preferred_element_type=f32)
        + bc2r)

    c = 1.4426950408889634 / _T
    aic = a_i * c
    pic = pc_i * c
    pjtc = pc_jt * c
    bjtc = b_jt * c
    dx12 = aic[:, 0:1] - pjtc[0:1, :]
    dy12 = aic[:, 1:2] - pjtc[1:2, :]
    dz12 = aic[:, 2:3] - pjtc[2:3, :]
    s12 = dx12 * dx12 + dy12 * dy12 + dz12 * dz12 + 1e-24
    d12 = s12 * jax.lax.rsqrt(s12)
    dx21 = pic[:, 0:1] - bjtc[0:1, :]
    dy21 = pic[:, 1:2] - bjtc[1:2, :]
    dz21 = pic[:, 2:3] - bjtc[2:3, :]
    s21 = dx21 * dx21 + dy21 * dy21 + dz21 * dz21 + 1e-24
    d21 = s21 * jax.lax.rsqrt(s21)

    u = w_j / (w_i + w_j)
    blend = d12 + (d21 - d12) * u
    mb = jnp.min(blend, axis=1, keepdims=True)
    e = jnp.exp2(mb - blend)
    out_ref[q, 0] = e * (1.0 / jnp.sum(e, axis=1, keepdims=True))


def kernel(xyz, Wf1, bf1, Wf2, bf2, Wc1, bc1, Wc2, bc2):
    vidx = jnp.asarray(_VIDX, dtype=jnp.int32)

    view = lambda col: pl.BlockSpec(
        (1, 1, _N_POINT, 3), lambda p, v: (0, v[p, col], 0, 0))
    full = lambda shape: pl.BlockSpec(shape, lambda p, v: (0,) * len(shape))
    grid_spec = pltpu.PrefetchScalarGridSpec(
        num_scalar_prefetch=1,
        grid=(3,),
        in_specs=[
            view(0), view(1), view(2), view(3),
            full((3, 64)), full((1, 64)), full((64, 1)),
            full((64, 3)), full((1, 3)), full((3, 1)),
            full((3, 64)), full((1, 64)), full((64, 1)),
            full((64, 1)), full((1, 1)),
        ],
        out_specs=pl.BlockSpec((2, 1, _N_POINT, _N_POINT),
                               lambda p, v: (p, 0, 0, 0)),
    )
    return pl.pallas_call(
        _pair_kernel,
        grid_spec=grid_spec,
        out_shape=jax.ShapeDtypeStruct((6, 1, _N_POINT, _N_POINT),
                                       jnp.float32),
        compiler_params=pltpu.CompilerParams(
            dimension_semantics=("arbitrary",)),
    )(
        vidx, xyz, xyz, xyz, xyz,
        Wf1, bf1.reshape(1, 64), bf1.reshape(64, 1),
        Wf2, bf2.reshape(1, 3), bf2.reshape(3, 1),
        Wc1, bc1.reshape(1, 64), bc1.reshape(64, 1),
        Wc2, bc2.reshape(1, 1),
    )


# speedup vs baseline: 1.5378x; 1.0745x over previous
import jax
import jax.numpy as jnp
from jax.experimental import pallas as pl
from jax.experimental.pallas import tpu as pltpu

_N_POINT = 1024
_T = 0.01
_VIDX = ((0, 1, 0, 2), (0, 3, 1, 2), (1, 3, 2, 3))


def _pair_kernel(vidx_ref, xi0_ref, xj0_ref, xi1_ref, xj1_ref,
                 wf1_ref, bf1r_ref, wf2_ref, bf2r_ref,
                 wc1_ref, bc1r_ref, wc2_ref, bc2r_ref,
                 out_ref):
    del vidx_ref
    wf1t = wf1_ref[...].T
    wf2t = wf2_ref[...].T
    wc1t = wc1_ref[...].T
    wc2t = wc2_ref[...].T
    bf1c = bf1r_ref[...].T
    bf2c = bf2r_ref[...].T
    bc1c = bc1r_ref[...].T
    for q, (xi_ref, xj_ref) in enumerate(((xi0_ref, xj0_ref),
                                          (xi1_ref, xj1_ref))):
        _one_pair(xi_ref[0, 0], xj_ref[0, 0],
                  wf1_ref[...], bf1r_ref[...], wf2_ref[...], bf2r_ref[...],
                  wc1_ref[...], bc1r_ref[...], wc2_ref[...], bc2r_ref[...],
                  wf1t, bf1c, wf2t, bf2c,
                  wc1t, bc1c, wc2t,
                  out_ref, q)


def _one_pair(pc_i, pc_j,
              wf1, bf1r, wf2, bf2r, wc1, bc1r, wc2, bc2r,
              wf1t, bf1c, wf2t, bf2c, wc1t, bc1c, wc2t,
              out_ref, q):
    f32 = jnp.float32
    pc_jt = pc_j.T

    h_i = jnp.tanh(jnp.dot(pc_i, wf1, preferred_element_type=f32)
                   + bf1r)
    a_i = pc_i + jnp.dot(h_i, wf2, preferred_element_type=f32) \
        + bf2r
    hc_i = jnp.tanh(jnp.dot(a_i, wc1, preferred_element_type=f32)
                    + bc1r)
    w_i = jax.nn.sigmoid(
        jnp.dot(hc_i, wc2, preferred_element_type=f32)
        + bc2r)

    h_jt = jnp.tanh(jnp.dot(wf1t, pc_jt, preferred_element_type=f32)
                    + bf1c)
    b_jt = pc_jt + jnp.dot(wf2t, h_jt, preferred_element_type=f32) \
        + bf2c
    hc_jt = jnp.tanh(jnp.dot(wc1t, b_jt, preferred_element_type=f32)
                     + bc1c)
    w_j = jax.nn.sigmoid(
        jnp.dot(wc2t, hc_jt, preferred_element_type=f32)
        + bc2r)

    c = 1.4426950408889634 / _T
    aic = a_i * c
    pic = pc_i * c
    pjtc = pc_jt * c
    bjtc = b_jt * c
    dx12 = aic[:, 0:1] - pjtc[0:1, :]
    dy12 = aic[:, 1:2] - pjtc[1:2, :]
    dz12 = aic[:, 2:3] - pjtc[2:3, :]
    s12 = dx12 * dx12 + dy12 * dy12 + dz12 * dz12 + 1e-24
    d12 = s12 * jax.lax.rsqrt(s12)
    dx21 = pic[:, 0:1] - bjtc[0:1, :]
    dy21 = pic[:, 1:2] - bjtc[1:2, :]
    dz21 = pic[:, 2:3] - bjtc[2:3, :]
    s21 = dx21 * dx21 + dy21 * dy21 + dz21 * dz21 + 1e-24
    d21 = s21 * jax.lax.rsqrt(s21)

    u = w_j / (w_i + w_j)
    blend = d12 + (d21 - d12) * u
    mb = jnp.min(blend, axis=1, keepdims=True)
    e = jnp.exp2(mb - blend)
    out_ref[q, 0] = e * (1.0 / jnp.sum(e, axis=1, keepdims=True))


def kernel(xyz, Wf1, bf1, Wf2, bf2, Wc1, bc1, Wc2, bc2):
    vidx = jnp.asarray(_VIDX, dtype=jnp.int32)

    view = lambda col: pl.BlockSpec(
        (1, 1, _N_POINT, 3), lambda p, v: (0, v[p, col], 0, 0))
    full = lambda shape: pl.BlockSpec(shape, lambda p, v: (0,) * len(shape))
    grid_spec = pltpu.PrefetchScalarGridSpec(
        num_scalar_prefetch=1,
        grid=(3,),
        in_specs=[
            view(0), view(1), view(2), view(3),
            full((3, 64)), full((1, 64)),
            full((64, 3)), full((1, 3)),
            full((3, 64)), full((1, 64)),
            full((64, 1)), full((1, 1)),
        ],
        out_specs=pl.BlockSpec((2, 1, _N_POINT, _N_POINT),
                               lambda p, v: (p, 0, 0, 0)),
    )
    return pl.pallas_call(
        _pair_kernel,
        grid_spec=grid_spec,
        out_shape=jax.ShapeDtypeStruct((6, 1, _N_POINT, _N_POINT),
                                       jnp.float32),
        compiler_params=pltpu.CompilerParams(
            dimension_semantics=("arbitrary",)),
    )(
        vidx, xyz, xyz, xyz, xyz,
        Wf1, bf1.reshape(1, 64),
        Wf2, bf2.reshape(1, 3),
        Wc1, bc1.reshape(1, 64),
        Wc2, bc2.reshape(1, 1),
    )


# speedup vs baseline: 1.5499x; 1.0079x over previous
import jax
import jax.numpy as jnp
from jax.experimental import pallas as pl
from jax.experimental.pallas import tpu as pltpu

_N_POINT = 1024
_T = 0.01
_VIDX = ((0, 1, 0, 2), (0, 3, 1, 2), (1, 3, 2, 3))


def _pair_kernel(vidx_ref, x_ref,
                 wf1_ref, bf1_ref, wf2_ref, bf2_ref,
                 wc1_ref, bc1_ref, wc2_ref, bc2_ref,
                 out_ref):
    p = pl.program_id(0)
    bf1r = bf1_ref[...].reshape(1, 64)
    bf2r = bf2_ref[...].reshape(1, 3)
    bc1r = bc1_ref[...].reshape(1, 64)
    bc2r = bc2_ref[...].reshape(1, 1)
    wf1t = wf1_ref[...].T
    wf2t = wf2_ref[...].T
    wc1t = wc1_ref[...].T
    wc2t = wc2_ref[...].T
    bf1c = bf1r.T
    bf2c = bf2r.T
    bc1c = bc1r.T
    for q in range(2):
        pc_i = x_ref[0, vidx_ref[p, 2 * q]]
        pc_j = x_ref[0, vidx_ref[p, 2 * q + 1]]
        _one_pair(pc_i, pc_j,
                  wf1_ref[...], bf1r, wf2_ref[...], bf2r,
                  wc1_ref[...], bc1r, wc2_ref[...], bc2r,
                  wf1t, bf1c, wf2t, bf2c,
                  wc1t, bc1c, wc2t,
                  out_ref, q)


def _one_pair(pc_i, pc_j,
              wf1, bf1r, wf2, bf2r, wc1, bc1r, wc2, bc2r,
              wf1t, bf1c, wf2t, bf2c, wc1t, bc1c, wc2t,
              out_ref, q):
    f32 = jnp.float32
    pc_jt = pc_j.T

    h_i = jnp.tanh(jnp.dot(pc_i, wf1, preferred_element_type=f32)
                   + bf1r)
    a_i = pc_i + jnp.dot(h_i, wf2, preferred_element_type=f32) \
        + bf2r
    hc_i = jnp.tanh(jnp.dot(a_i, wc1, preferred_element_type=f32)
                    + bc1r)
    w_i = jax.nn.sigmoid(
        jnp.dot(hc_i, wc2, preferred_element_type=f32)
        + bc2r)

    h_jt = jnp.tanh(jnp.dot(wf1t, pc_jt, preferred_element_type=f32)
                    + bf1c)
    b_jt = pc_jt + jnp.dot(wf2t, h_jt, preferred_element_type=f32) \
        + bf2c
    hc_jt = jnp.tanh(jnp.dot(wc1t, b_jt, preferred_element_type=f32)
                     + bc1c)
    w_j = jax.nn.sigmoid(
        jnp.dot(wc2t, hc_jt, preferred_element_type=f32)
        + bc2r)

    c = 1.4426950408889634 / _T
    aic = a_i * c
    pic = pc_i * c
    pjtc = pc_jt * c
    bjtc = b_jt * c
    dx12 = aic[:, 0:1] - pjtc[0:1, :]
    dy12 = aic[:, 1:2] - pjtc[1:2, :]
    dz12 = aic[:, 2:3] - pjtc[2:3, :]
    s12 = dx12 * dx12 + dy12 * dy12 + dz12 * dz12 + 1e-24
    d12 = s12 * jax.lax.rsqrt(s12)
    dx21 = pic[:, 0:1] - bjtc[0:1, :]
    dy21 = pic[:, 1:2] - bjtc[1:2, :]
    dz21 = pic[:, 2:3] - bjtc[2:3, :]
    s21 = dx21 * dx21 + dy21 * dy21 + dz21 * dz21 + 1e-24
    d21 = s21 * jax.lax.rsqrt(s21)

    u = w_j / (w_i + w_j)
    blend = d12 + (d21 - d12) * u
    mb = jnp.min(blend, axis=1, keepdims=True)
    e = jnp.exp2(mb - blend)
    out_ref[q, 0] = e * (1.0 / jnp.sum(e, axis=1, keepdims=True))


def kernel(xyz, Wf1, bf1, Wf2, bf2, Wc1, bc1, Wc2, bc2):
    vidx = jnp.asarray(_VIDX, dtype=jnp.int32)

    full = lambda shape: pl.BlockSpec(shape, lambda p, v: (0,) * len(shape))
    grid_spec = pltpu.PrefetchScalarGridSpec(
        num_scalar_prefetch=1,
        grid=(3,),
        in_specs=[
            full((1, 4, _N_POINT, 3)),
            full((3, 64)), full((64,)),
            full((64, 3)), full((3,)),
            full((3, 64)), full((64,)),
            full((64, 1)), full((1,)),
        ],
        out_specs=pl.BlockSpec((2, 1, _N_POINT, _N_POINT),
                               lambda p, v: (p, 0, 0, 0)),
    )
    return pl.pallas_call(
        _pair_kernel,
        grid_spec=grid_spec,
        out_shape=jax.ShapeDtypeStruct((6, 1, _N_POINT, _N_POINT),
                                       jnp.float32),
        compiler_params=pltpu.CompilerParams(
            dimension_semantics=("arbitrary",)),
    )(
        vidx, xyz,
        Wf1, bf1, Wf2, bf2, Wc1, bc1, Wc2, bc2,
    )


# speedup vs baseline: 1.5929x; 1.0278x over previous
import jax
import jax.numpy as jnp
from jax.experimental import pallas as pl
from jax.experimental.pallas import tpu as pltpu

_N_POINT = 1024
_T = 0.01
_VIDX = ((0, 1, 0, 2, 0, 3), (1, 2, 1, 3, 2, 3))


def _pair_kernel(vidx_ref, x_ref,
                 wf1_ref, bf1_ref, wf2_ref, bf2_ref,
                 wc1_ref, bc1_ref, wc2_ref, bc2_ref,
                 out_ref):
    p = pl.program_id(0)
    bf1r = bf1_ref[...].reshape(1, 64)
    bf2r = bf2_ref[...].reshape(1, 3)
    bc1r = bc1_ref[...].reshape(1, 64)
    bc2r = bc2_ref[...].reshape(1, 1)
    wf1t = wf1_ref[...].T
    wf2t = wf2_ref[...].T
    wc1t = wc1_ref[...].T
    wc2t = wc2_ref[...].T
    bf1c = bf1r.T
    bf2c = bf2r.T
    bc1c = bc1r.T
    for q in range(3):
        pc_i = x_ref[0, vidx_ref[p, 2 * q]]
        pc_j = x_ref[0, vidx_ref[p, 2 * q + 1]]
        _one_pair(pc_i, pc_j,
                  wf1_ref[...], bf1r, wf2_ref[...], bf2r,
                  wc1_ref[...], bc1r, wc2_ref[...], bc2r,
                  wf1t, bf1c, wf2t, bf2c,
                  wc1t, bc1c, wc2t,
                  out_ref, q)


def _one_pair(pc_i, pc_j,
              wf1, bf1r, wf2, bf2r, wc1, bc1r, wc2, bc2r,
              wf1t, bf1c, wf2t, bf2c, wc1t, bc1c, wc2t,
              out_ref, q):
    f32 = jnp.float32
    pc_jt = pc_j.T

    h_i = jnp.tanh(jnp.dot(pc_i, wf1, preferred_element_type=f32)
                   + bf1r)
    a_i = pc_i + jnp.dot(h_i, wf2, preferred_element_type=f32) \
        + bf2r
    hc_i = jnp.tanh(jnp.dot(a_i, wc1, preferred_element_type=f32)
                    + bc1r)
    w_i = jax.nn.sigmoid(
        jnp.dot(hc_i, wc2, preferred_element_type=f32)
        + bc2r)

    h_jt = jnp.tanh(jnp.dot(wf1t, pc_jt, preferred_element_type=f32)
                    + bf1c)
    b_jt = pc_jt + jnp.dot(wf2t, h_jt, preferred_element_type=f32) \
        + bf2c
    hc_jt = jnp.tanh(jnp.dot(wc1t, b_jt, preferred_element_type=f32)
                     + bc1c)
    w_j = jax.nn.sigmoid(
        jnp.dot(wc2t, hc_jt, preferred_element_type=f32)
        + bc2r)

    c = 1.4426950408889634 / _T
    aic = a_i * c
    pic = pc_i * c
    pjtc = pc_jt * c
    bjtc = b_jt * c
    dx12 = aic[:, 0:1] - pjtc[0:1, :]
    dy12 = aic[:, 1:2] - pjtc[1:2, :]
    dz12 = aic[:, 2:3] - pjtc[2:3, :]
    s12 = (dx12 * dx12 + dy12 * dy12) + (dz12 * dz12 + 1e-24)
    d12 = s12 * jax.lax.rsqrt(s12)
    dx21 = pic[:, 0:1] - bjtc[0:1, :]
    dy21 = pic[:, 1:2] - bjtc[1:2, :]
    dz21 = pic[:, 2:3] - bjtc[2:3, :]
    s21 = (dx21 * dx21 + dy21 * dy21) + (dz21 * dz21 + 1e-24)
    d21 = s21 * jax.lax.rsqrt(s21)

    u = w_j / (w_i + w_j)
    blend = d12 + (d21 - d12) * u
    mb = jnp.min(blend, axis=1, keepdims=True)
    e = jnp.exp2(mb - blend)
    out_ref[q, 0] = e * (1.0 / jnp.sum(e, axis=1, keepdims=True))


def kernel(xyz, Wf1, bf1, Wf2, bf2, Wc1, bc1, Wc2, bc2):
    vidx = jnp.asarray(_VIDX, dtype=jnp.int32)

    full = lambda shape: pl.BlockSpec(shape, lambda p, v: (0,) * len(shape))
    grid_spec = pltpu.PrefetchScalarGridSpec(
        num_scalar_prefetch=1,
        grid=(2,),
        in_specs=[
            full((1, 4, _N_POINT, 3)),
            full((3, 64)), full((64,)),
            full((64, 3)), full((3,)),
            full((3, 64)), full((64,)),
            full((64, 1)), full((1,)),
        ],
        out_specs=pl.BlockSpec((3, 1, _N_POINT, _N_POINT),
                               lambda p, v: (p, 0, 0, 0)),
    )
    return pl.pallas_call(
        _pair_kernel,
        grid_spec=grid_spec,
        out_shape=jax.ShapeDtypeStruct((6, 1, _N_POINT, _N_POINT),
                                       jnp.float32),
        compiler_params=pltpu.CompilerParams(
            dimension_semantics=("arbitrary",)),
    )(
        vidx, xyz,
        Wf1, bf1, Wf2, bf2, Wc1, bc1, Wc2, bc2,
    )


# speedup vs baseline: 1.6022x; 1.0058x over previous
import jax
import jax.numpy as jnp
from jax.experimental import pallas as pl
from jax.experimental.pallas import tpu as pltpu

_N_POINT = 1024
_T = 0.01
_VIDX = ((0, 1, 0, 2, 0, 3), (1, 2, 1, 3, 2, 3))


def _pair_kernel(vidx_ref, x_ref,
                 wf1_ref, bf1_ref, wf2_ref, bf2_ref,
                 wc1_ref, bc1_ref, wc2_ref, bc2_ref,
                 out_ref):
    p = pl.program_id(0)
    bf1r = bf1_ref[...].reshape(1, 64)
    bf2r = bf2_ref[...].reshape(1, 3)
    bc1r = bc1_ref[...].reshape(1, 64)
    bc2r = bc2_ref[...].reshape(1, 1)
    wf1t = wf1_ref[...].T
    wf2t = wf2_ref[...].T
    wc1t = wc1_ref[...].T
    wc2t = wc2_ref[...].T
    bf1c = bf1r.T
    bf2c = bf2r.T
    bc1c = bc1r.T
    for q in range(3):
        pc_i = x_ref[0, vidx_ref[p, 2 * q]]
        pc_j = x_ref[0, vidx_ref[p, 2 * q + 1]]
        _one_pair(pc_i, pc_j,
                  wf1_ref[...], bf1r, wf2_ref[...], bf2r,
                  wc1_ref[...], bc1r, wc2_ref[...], bc2r,
                  wf1t, bf1c, wf2t, bf2c,
                  wc1t, bc1c, wc2t,
                  out_ref, q)


def _one_pair(pc_i, pc_j,
              wf1, bf1r, wf2, bf2r, wc1, bc1r, wc2, bc2r,
              wf1t, bf1c, wf2t, bf2c, wc1t, bc1c, wc2t,
              out_ref, q):
    f32 = jnp.float32
    pc_jt = pc_j.T

    h_i = jnp.tanh(jnp.dot(pc_i, wf1, preferred_element_type=f32)
                   + bf1r)
    a_i = pc_i + jnp.dot(h_i, wf2, preferred_element_type=f32) \
        + bf2r
    hc_i = jnp.tanh(jnp.dot(a_i, wc1, preferred_element_type=f32)
                    + bc1r)
    w_i = jax.nn.sigmoid(
        jnp.dot(hc_i, wc2, preferred_element_type=f32)
        + bc2r)

    h_jt = jnp.tanh(jnp.dot(wf1t, pc_jt, preferred_element_type=f32)
                    + bf1c)
    b_jt = pc_jt + jnp.dot(wf2t, h_jt, preferred_element_type=f32) \
        + bf2c
    hc_jt = jnp.tanh(jnp.dot(wc1t, b_jt, preferred_element_type=f32)
                     + bc1c)
    w_j = jax.nn.sigmoid(
        jnp.dot(wc2t, hc_jt, preferred_element_type=f32)
        + bc2r)

    c = 1.4426950408889634 / _T
    aic = a_i * c
    pic = pc_i * c
    pjtc = pc_jt * c
    bjtc = b_jt * c
    dx12 = aic[:, 0:1] - pjtc[0:1, :]
    dy12 = aic[:, 1:2] - pjtc[1:2, :]
    dz12 = aic[:, 2:3] - pjtc[2:3, :]
    s12 = (dx12 * dx12 + dy12 * dy12) + (dz12 * dz12 + 1e-24)
    d12 = s12 * jax.lax.rsqrt(s12)
    dx21 = pic[:, 0:1] - bjtc[0:1, :]
    dy21 = pic[:, 1:2] - bjtc[1:2, :]
    dz21 = pic[:, 2:3] - bjtc[2:3, :]
    s21 = (dx21 * dx21 + dy21 * dy21) + (dz21 * dz21 + 1e-24)
    d21 = s21 * jax.lax.rsqrt(s21)

    u = w_j / (w_i + w_j)
    blend = d12 + (d21 - d12) * u
    mb = jnp.min(blend, axis=1, keepdims=True)
    e = jnp.exp2(mb - blend)
    out_ref[q, 0] = e * (1.0 / jnp.sum(e, axis=1, keepdims=True))


def kernel(xyz, Wf1, bf1, Wf2, bf2, Wc1, bc1, Wc2, bc2):
    vidx = jnp.asarray(_VIDX, dtype=jnp.int32)

    full = lambda shape: pl.BlockSpec(shape, lambda p, v: (0,) * len(shape))
    grid_spec = pltpu.PrefetchScalarGridSpec(
        num_scalar_prefetch=1,
        grid=(2,),
        in_specs=[
            full((1, 4, _N_POINT, 3)),
            full((3, 64)), full((64,)),
            full((64, 3)), full((3,)),
            full((3, 64)), full((64,)),
            full((64, 1)), full((1,)),
        ],
        out_specs=pl.BlockSpec((3, 1, _N_POINT, _N_POINT),
                               lambda p, v: (p, 0, 0, 0)),
    )
    return pl.pallas_call(
        _pair_kernel,
        grid_spec=grid_spec,
        out_shape=jax.ShapeDtypeStruct((6, 1, _N_POINT, _N_POINT),
                                       jnp.float32),
        compiler_params=pltpu.CompilerParams(
            dimension_semantics=("parallel",)),
    )(
        vidx, xyz,
        Wf1, bf1, Wf2, bf2, Wc1, bc1, Wc2, bc2,
    )
